# baseline (speedup 1.0000x reference)
import jax
import jax.numpy as jnp
from jax.experimental import pallas as pl
from jax.experimental.pallas import tpu as pltpu

B = 32
N = 128
K = 128
NODE = 128
BT = 8
TC = 8
ALPHA = 0.2
H = 150
PH = 256
HB = 16
BF = jnp.bfloat16
F32 = jnp.float32


def _mm(a, b_bf, dims=(((1,), (0,)), ((), ()))):
    return jax.lax.dot_general(a.astype(BF), b_bf, dims,
                               preferred_element_type=F32)


def _count_kernel(sf_ref, df_ref, st_ref, dt_ref,
                  cf_ref, lcf_ref, mf_ref, ct_ref, lct_ref, mt_ref):
    c = pl.program_id(0)
    nc = pl.num_programs(0)

    @pl.when(c == 0)
    def _():
        cf_ref[...] = jnp.zeros_like(cf_ref)
        ct_ref[...] = jnp.zeros_like(ct_ref)

    def onehot_t(idx_row):
        ch = idx_row.shape[-1]
        lanes = jax.lax.broadcasted_iota(jnp.int32, (NODE, ch), 0)
        return (lanes == jnp.broadcast_to(idx_row, (NODE, ch))).astype(BF)

    dn = (((1,), (1,)), ((), ()))
    cf_ref[...] += jax.lax.dot_general(
        onehot_t(sf_ref[0]), onehot_t(df_ref[0]), dn,
        preferred_element_type=F32)
    ct_ref[...] += jax.lax.dot_general(
        onehot_t(st_ref[0]), onehot_t(dt_ref[0]), dn,
        preferred_element_type=F32)

    @pl.when(c == nc - 1)
    def _():
        ninf = jnp.float32(-jnp.inf)
        cf = cf_ref[...]
        ct = ct_ref[...]
        lcf_ref[...] = jnp.where(cf > 0, jnp.log(cf), ninf)
        lct_ref[...] = jnp.where(ct > 0, jnp.log(ct), ninf)
        mf_ref[...] = jnp.where(cf > 0, 0.0, ninf)
        mt_ref[...] = jnp.where(ct > 0, 0.0, ninf)


def _build_counts(fc_ei, tc_ei):
    nb, ch = 8, fc_ei.shape[1] // 8
    args = [a.reshape(nb, 1, ch) for a in (fc_ei[0], fc_ei[1], tc_ei[0], tc_ei[1])]
    spec = pl.BlockSpec((1, 1, ch), lambda c: (c, 0, 0))
    ospec = pl.BlockSpec((NODE, NODE), lambda c: (0, 0))
    return pl.pallas_call(
        _count_kernel,
        grid=(nb,),
        in_specs=[spec] * 4,
        out_specs=[ospec] * 6,
        out_shape=[jax.ShapeDtypeStruct((NODE, NODE), F32)] * 6,
    )(*args)


def _gat_batch(HC, logC, minf, W_bf, a2, bias):
    dn0 = (((0,), (0,)), ((), ()))
    dnT = (((1,), (1,)), ((), ()))
    h1 = jax.lax.dot_general(HC.astype(BF), W_bf, dn0,
                             preferred_element_type=F32)
    s = jax.lax.dot_general(h1, a2[0:1, :], dnT,
                            precision=jax.lax.Precision.DEFAULT,
                            preferred_element_type=F32)
    d = jax.lax.dot_general(a2[1:2, :], h1, dnT,
                            precision=jax.lax.Precision.DEFAULT,
                            preferred_element_type=F32)
    e = s.reshape(BT, NODE, 1) + d.reshape(BT, 1, NODE)
    e = jnp.maximum(e, ALPHA * e)
    emax = jnp.max(e + minf[None], axis=1, keepdims=True)
    emax = jnp.where(jnp.isfinite(emax), emax, 0.0)
    p = jnp.exp(e - emax + logC[None])
    denom = jnp.sum(p, axis=1, keepdims=True)
    alpha = (p * (1.0 / (denom + 1e-16))).astype(BF)
    h1b = h1.astype(BF)
    parts = [
        jax.lax.dot_general(alpha[j], h1b[j * NODE:(j + 1) * NODE], dn0,
                            preferred_element_type=F32)
        for j in range(BT)
    ]
    oc = jnp.concatenate(parts, axis=1)
    btile = jnp.broadcast_to(bias.reshape(1, 1, K), (1, BT, K)).reshape(1, BT * K)
    return jnp.maximum(oc + btile, 0.0)


def _branch_kernel(xpb_ref, xpf_ref, wc_ref, bc_ref,
                   lcf_ref, mf_ref, lct_ref, mt_ref,
                   gw_ref, ga_ref, gb_ref, out_ref):
    br = pl.program_id(0)
    wc = wc_ref[0]
    gw = gw_ref[0]
    ga = ga_ref[0]
    gb = gb_ref[0]

    y = bc_ref[0]
    for dd in range(7):
        a_d = jnp.concatenate([xpb_ref[j, dd:dd + N, :] for j in range(BT)],
                              axis=0)
        y = y + jax.lax.dot_general(a_d, wc[dd], (((1,), (0,)), ((), ())),
                                    preferred_element_type=F32)
    y = jnp.maximum(y, 0.0)
    yc = jnp.concatenate([y[j * N:(j + 1) * N] for j in range(BT)], axis=1)
    xc = jnp.concatenate([xpf_ref[j, 3:3 + N, :] for j in range(BT)], axis=1)
    hc = jnp.where(br == 0, xc, yc)

    lcf = lcf_ref[...]
    mf = mf_ref[...]
    lct = lct_ref[...]
    mt = mt_ref[...]
    for layer in range(2):
        fc = _gat_batch(hc, lcf, mf, gw[layer, 0], ga[layer, 0],
                        gb[layer, 0])
        tc = _gat_batch(fc, lct, mt, gw[layer, 1], ga[layer, 1],
                        gb[layer, 1])
        hc = hc + tc
    for j in range(BT):
        out_ref[0, j] = hc[:, j * K:(j + 1) * K]


def _run_branches(xpad_bf, xpad, Wc, bc, lcf, mf, lct, mt, gat_W, gat_a, gat_b):
    full = lambda shape: pl.BlockSpec(shape, lambda r, c: tuple(0 for _ in shape))
    return pl.pallas_call(
        _branch_kernel,
        grid=(3, B // BT),
        in_specs=[
            pl.BlockSpec((BT, 136, K), lambda r, c: (c, 0, 0)),
            pl.BlockSpec((BT, 136, K), lambda r, c: (c, 0, 0)),
            pl.BlockSpec((1, 7, K, K), lambda r, c: (r, 0, 0, 0)),
            pl.BlockSpec((1, 1, K), lambda r, c: (r, 0, 0)),
            full((NODE, NODE)),
            full((NODE, NODE)),
            full((NODE, NODE)),
            full((NODE, NODE)),
            pl.BlockSpec((1, 2, 2, K, K), lambda r, c: (r, 0, 0, 0, 0)),
            pl.BlockSpec((1, 2, 2, 2, K), lambda r, c: (r, 0, 0, 0, 0)),
            pl.BlockSpec((1, 2, 2, K), lambda r, c: (r, 0, 0, 0)),
        ],
        out_specs=pl.BlockSpec((1, BT, N, K), lambda r, c: (r, c, 0, 0)),
        out_shape=jax.ShapeDtypeStruct((3, B, N, K), F32),
        compiler_params=pltpu.CompilerParams(
            dimension_semantics=("parallel", "parallel")),
    )(xpad_bf, xpad, Wc, bc, lcf, mf, lct, mt, gat_W, gat_a, gat_b)


def _rnn_kernel(hct_ref, wif_ref, whf_ref, bf_ref, wib_ref, bb_ref,
                wig_ref, big_ref, whg_ref, bhg_ref, wfc_ref, bfc_ref,
                out_ref, xp_ref, hall_ref):
    wif = wif_ref[...]
    bf = bf_ref[...]
    for tc in range(N // TC):
        blk = hct_ref[tc * TC:(tc + 1) * TC]
        a = blk.reshape(TC * HB, 3 * K)
        xp_ref[tc * TC:(tc + 1) * TC] = (
            _mm(a, wif) + bf).reshape(TC, HB, 4 * PH)

    whf = whf_ref[...]

    def lstm_step4(i4, carry):
        h, c = carry
        for u in range(4):
            g = xp_ref[i4 * 4 + u] + _mm(h, whf)
            i_g = jax.nn.sigmoid(g[:, 0:PH])
            f_g = jax.nn.sigmoid(g[:, PH:2 * PH])
            g_g = jnp.tanh(g[:, 2 * PH:3 * PH])
            o_g = jax.nn.sigmoid(g[:, 3 * PH:4 * PH])
            c = f_g * c + i_g * g_g
            h = o_g * jnp.tanh(c)
        return h, c

    z = jnp.zeros((HB, PH), F32)
    hf, _ = jax.lax.fori_loop(0, N // 4, lstm_step4, (z, z))

    gb = _mm(hct_ref[N - 1], wib_ref[...]) + bb_ref[...]
    cb = jax.nn.sigmoid(gb[:, 0:PH]) * jnp.tanh(gb[:, 2 * PH:3 * PH])
    hb = jax.nn.sigmoid(gb[:, 3 * PH:4 * PH]) * jnp.tanh(cb)

    hend = jnp.concatenate([hf, hb], axis=1)
    gi = _mm(hend, wig_ref[...]) + big_ref[...]

    whg = whg_ref[...]
    bhg = bhg_ref[...]

    def gru_step4(i4, h):
        for u in range(4):
            gh = _mm(h, whg) + bhg
            r = jax.nn.sigmoid(gi[:, 0:PH] + gh[:, 0:PH])
            zg = jax.nn.sigmoid(gi[:, PH:2 * PH] + gh[:, PH:2 * PH])
            nc = jnp.tanh(gi[:, 2 * PH:3 * PH] + r * gh[:, 2 * PH:3 * PH])
            h = (1.0 - zg) * nc + zg * h
            hall_ref[i4 * 4 + u] = h
        return h

    jax.lax.fori_loop(0, N // 4, gru_step4, z)

    wfc = wfc_ref[...]
    bfc = bfc_ref[...]
    for tc in range(N // TC):
        blk = hall_ref[tc * TC:(tc + 1) * TC]
        a = blk.reshape(TC * HB, PH)
        out_ref[tc * TC:(tc + 1) * TC] = (
            _mm(a, wfc) + bfc).reshape(TC, HB, K)


def _run_rnn(hct, wif, whf, bf, wib, bb, wig, big, whg, bhg, wfc, bfc):
    full = lambda a: pl.BlockSpec(a.shape, lambda g: tuple(0 for _ in a.shape))
    args = (hct, wif, whf, bf, wib, bb, wig, big, whg, bhg, wfc, bfc)
    specs = [full(a) for a in args]
    specs[0] = pl.BlockSpec((N, HB, 3 * K), lambda g: (0, g, 0))
    return pl.pallas_call(
        _rnn_kernel,
        grid=(B // HB,),
        in_specs=specs,
        out_specs=pl.BlockSpec((N, HB, K), lambda g: (0, g, 0)),
        out_shape=jax.ShapeDtypeStruct((N, B, K), F32),
        scratch_shapes=[pltpu.VMEM((N, HB, 4 * PH), F32),
                        pltpu.VMEM((N, HB, PH), F32)],
        compiler_params=pltpu.CompilerParams(
            dimension_semantics=("parallel",)),
    )(*args)


def _pad_gates(w_t, n_gates, in_rows):
    in_dim = w_t.shape[1]
    out = jnp.zeros((in_rows, n_gates * PH), F32)
    for g in range(n_gates):
        out = out.at[0:in_dim, g * PH:g * PH + H].set(w_t[g * H:(g + 1) * H, :].T)
    return out


def _pad_bias(b, n_gates):
    out = jnp.zeros((1, n_gates * PH), F32)
    for g in range(n_gates):
        out = out.at[0, g * PH:g * PH + H].set(b[g * H:(g + 1) * H])
    return out


def kernel(x, fc_edge_index, tc_edge_index, conv2_W, conv2_b, conv3_W, conv3_b,
           gat_W, gat_a, gat_b, lstm_W_ih, lstm_W_hh, lstm_b_ih, lstm_b_hh,
           gru_W_ih, gru_W_hh, gru_b_ih, gru_b_hh, fc_W, fc_b):
    fc_ei = fc_edge_index[-1].astype(jnp.int32)
    tc_ei = tc_edge_index[-1].astype(jnp.int32)

    _, lcf, mf, _, lct, mt = _build_counts(fc_ei, tc_ei)

    xpad = jnp.pad(x, ((0, 0), (3, 5), (0, 0)))
    Wc = jnp.zeros((3, 7, K, K), F32)
    Wc = Wc.at[0, 3].set(jnp.eye(K, dtype=F32))
    for d in range(5):
        Wc = Wc.at[1, d + 1].set(conv2_W[:, :, d].T)
    for d in range(7):
        Wc = Wc.at[2, d].set(conv3_W[:, :, d].T)
    bc = jnp.stack([jnp.zeros_like(conv2_b), conv2_b, conv3_b]).reshape(3, 1, K)
    hs = _run_branches(xpad.astype(BF), xpad, Wc.astype(BF), bc,
                       lcf, mf, lct, mt, gat_W.astype(BF), gat_a, gat_b)

    hct = hs.transpose(2, 1, 0, 3).reshape(N, B, 3 * K)
    wif = _pad_gates(lstm_W_ih[0], 4, 3 * K)
    whf = _pad_gates(lstm_W_hh[0], 4, PH)
    bf = _pad_bias(lstm_b_ih[0] + lstm_b_hh[0], 4)
    wib = _pad_gates(lstm_W_ih[1], 4, 3 * K)
    bb = _pad_bias(lstm_b_ih[1] + lstm_b_hh[1], 4)
    wig = jnp.zeros((2 * PH, 3 * PH), F32)
    for g in range(3):
        blk = gru_W_ih[g * H:(g + 1) * H, :]
        wig = wig.at[0:H, g * PH:g * PH + H].set(blk[:, 0:H].T)
        wig = wig.at[PH:PH + H, g * PH:g * PH + H].set(blk[:, H:2 * H].T)
    big = _pad_bias(gru_b_ih, 3)
    whg = _pad_gates(gru_W_hh, 3, PH)
    bhg = _pad_bias(gru_b_hh, 3)
    wfc = jnp.zeros((PH, K), F32).at[0:H, :].set(fc_W.T)
    bfc = fc_b.reshape(1, K)
    outt = _run_rnn(hct, wif.astype(BF), whf.astype(BF), bf, wib.astype(BF),
                    bb, wig.astype(BF), big, whg.astype(BF), bhg,
                    wfc.astype(BF), bfc)
    return outt.transpose(1, 0, 2)


# speedup vs baseline: 1.1684x; 1.1684x over previous
import jax
import jax.numpy as jnp
from jax.experimental import pallas as pl
from jax.experimental.pallas import tpu as pltpu

B = 32
N = 128
K = 128
NODE = 128
BT = 8
TC = 8
ALPHA = 0.2
H = 150
PH = 256
BF = jnp.bfloat16
F32 = jnp.float32


def _mm(a, b_bf, dims=(((1,), (0,)), ((), ()))):
    return jax.lax.dot_general(a.astype(BF), b_bf, dims,
                               preferred_element_type=F32)


def _count_kernel(sf_ref, df_ref, st_ref, dt_ref,
                  cf_ref, lcf_ref, mf_ref, ct_ref, lct_ref, mt_ref):
    c = pl.program_id(0)
    nc = pl.num_programs(0)

    @pl.when(c == 0)
    def _():
        cf_ref[...] = jnp.zeros_like(cf_ref)
        ct_ref[...] = jnp.zeros_like(ct_ref)

    def onehot_t(idx_row):
        ch = idx_row.shape[-1]
        lanes = jax.lax.broadcasted_iota(jnp.int32, (NODE, ch), 0)
        return (lanes == jnp.broadcast_to(idx_row, (NODE, ch))).astype(BF)

    dn = (((1,), (1,)), ((), ()))
    cf_ref[...] += jax.lax.dot_general(
        onehot_t(sf_ref[0]), onehot_t(df_ref[0]), dn,
        preferred_element_type=F32)
    ct_ref[...] += jax.lax.dot_general(
        onehot_t(st_ref[0]), onehot_t(dt_ref[0]), dn,
        preferred_element_type=F32)

    @pl.when(c == nc - 1)
    def _():
        ninf = jnp.float32(-jnp.inf)
        cf = cf_ref[...]
        ct = ct_ref[...]
        lcf_ref[...] = jnp.where(cf > 0, jnp.log(cf), ninf)
        lct_ref[...] = jnp.where(ct > 0, jnp.log(ct), ninf)
        mf_ref[...] = jnp.where(cf > 0, 0.0, ninf)
        mt_ref[...] = jnp.where(ct > 0, 0.0, ninf)


def _build_counts(fc_ei, tc_ei):
    nb, ch = 8, fc_ei.shape[1] // 8
    args = [a.reshape(nb, 1, ch) for a in (fc_ei[0], fc_ei[1], tc_ei[0], tc_ei[1])]
    spec = pl.BlockSpec((1, 1, ch), lambda c: (c, 0, 0))
    ospec = pl.BlockSpec((NODE, NODE), lambda c: (0, 0))
    return pl.pallas_call(
        _count_kernel,
        grid=(nb,),
        in_specs=[spec] * 4,
        out_specs=[ospec] * 6,
        out_shape=[jax.ShapeDtypeStruct((NODE, NODE), F32)] * 6,
    )(*args)


def _gat_batch(HC, logC, minf, W_bf, a2, bias):
    dn0 = (((0,), (0,)), ((), ()))
    dnT = (((1,), (1,)), ((), ()))
    h1 = jax.lax.dot_general(HC.astype(BF), W_bf, dn0,
                             preferred_element_type=F32)
    s = jax.lax.dot_general(h1, a2[0:1, :], dnT,
                            precision=jax.lax.Precision.DEFAULT,
                            preferred_element_type=F32)
    d = jax.lax.dot_general(a2[1:2, :], h1, dnT,
                            precision=jax.lax.Precision.DEFAULT,
                            preferred_element_type=F32)
    e = s.reshape(BT, NODE, 1) + d.reshape(BT, 1, NODE)
    e = jnp.maximum(e, ALPHA * e)
    emax = jnp.max(e + minf[None], axis=1, keepdims=True)
    emax = jnp.where(jnp.isfinite(emax), emax, 0.0)
    p = jnp.exp(e - emax + logC[None])
    denom = jnp.sum(p, axis=1, keepdims=True)
    alpha = (p * (1.0 / (denom + 1e-16))).astype(BF)
    h1b = h1.astype(BF)
    parts = [
        jax.lax.dot_general(alpha[j], h1b[j * NODE:(j + 1) * NODE], dn0,
                            preferred_element_type=F32)
        for j in range(BT)
    ]
    oc = jnp.concatenate(parts, axis=1)
    btile = jnp.broadcast_to(bias.reshape(1, 1, K), (1, BT, K)).reshape(1, BT * K)
    return jnp.maximum(oc + btile, 0.0)


def _branch_kernel(xpb_ref, xpf_ref, wc_ref, bc_ref,
                   lcf_ref, mf_ref, lct_ref, mt_ref,
                   gw_ref, ga_ref, gb_ref, out_ref):
    br = pl.program_id(0)
    wc = wc_ref[0]
    gw = gw_ref[0]
    ga = ga_ref[0]
    gb = gb_ref[0]

    y = bc_ref[0]
    for dd in range(7):
        a_d = jnp.concatenate([xpb_ref[j, dd:dd + N, :] for j in range(BT)],
                              axis=0)
        y = y + jax.lax.dot_general(a_d, wc[dd], (((1,), (0,)), ((), ())),
                                    preferred_element_type=F32)
    y = jnp.maximum(y, 0.0)
    yc = jnp.concatenate([y[j * N:(j + 1) * N] for j in range(BT)], axis=1)
    xc = jnp.concatenate([xpf_ref[j, 3:3 + N, :] for j in range(BT)], axis=1)
    hc = jnp.where(br == 0, xc, yc)

    lcf = lcf_ref[...]
    mf = mf_ref[...]
    lct = lct_ref[...]
    mt = mt_ref[...]
    for layer in range(2):
        fc = _gat_batch(hc, lcf, mf, gw[layer, 0], ga[layer, 0],
                        gb[layer, 0])
        tc = _gat_batch(fc, lct, mt, gw[layer, 1], ga[layer, 1],
                        gb[layer, 1])
        hc = hc + tc
    for j in range(BT):
        out_ref[0, j] = hc[:, j * K:(j + 1) * K]


def _run_branches(xpad_bf, xpad, Wc, bc, lcf, mf, lct, mt, gat_W, gat_a, gat_b):
    full = lambda shape: pl.BlockSpec(shape, lambda r, c: tuple(0 for _ in shape))
    return pl.pallas_call(
        _branch_kernel,
        grid=(3, B // BT),
        in_specs=[
            pl.BlockSpec((BT, 136, K), lambda r, c: (c, 0, 0)),
            pl.BlockSpec((BT, 136, K), lambda r, c: (c, 0, 0)),
            pl.BlockSpec((1, 7, K, K), lambda r, c: (r, 0, 0, 0)),
            pl.BlockSpec((1, 1, K), lambda r, c: (r, 0, 0)),
            full((NODE, NODE)),
            full((NODE, NODE)),
            full((NODE, NODE)),
            full((NODE, NODE)),
            pl.BlockSpec((1, 2, 2, K, K), lambda r, c: (r, 0, 0, 0, 0)),
            pl.BlockSpec((1, 2, 2, 2, K), lambda r, c: (r, 0, 0, 0, 0)),
            pl.BlockSpec((1, 2, 2, K), lambda r, c: (r, 0, 0, 0)),
        ],
        out_specs=pl.BlockSpec((1, BT, N, K), lambda r, c: (r, c, 0, 0)),
        out_shape=jax.ShapeDtypeStruct((3, B, N, K), F32),
        compiler_params=pltpu.CompilerParams(
            dimension_semantics=("parallel", "parallel")),
    )(xpad_bf, xpad, Wc, bc, lcf, mf, lct, mt, gat_W, gat_a, gat_b)


def _rnn_kernel(hct_ref, wif_ref, whf_ref, bf_ref, wib_ref, bb_ref,
                wig_ref, big_ref, whg_ref, bhg_ref, wfc_ref, bfc_ref,
                out_ref, xp_ref, hall_ref):
    wif = wif_ref[...]
    bf = bf_ref[...]
    for tc in range(N // TC):
        blk = hct_ref[tc * TC:(tc + 1) * TC]
        a = blk.reshape(TC * B, 3 * K)
        xp_ref[tc * TC:(tc + 1) * TC] = (
            _mm(a, wif) + bf).reshape(TC, B, 4 * PH)

    whf = whf_ref[...]

    def lstm_step4(i4, carry):
        h, c = carry
        for u in range(4):
            g = xp_ref[i4 * 4 + u] + _mm(h, whf)
            i_g = jax.nn.sigmoid(g[:, 0:PH])
            f_g = jax.nn.sigmoid(g[:, PH:2 * PH])
            g_g = jnp.tanh(g[:, 2 * PH:3 * PH])
            o_g = jax.nn.sigmoid(g[:, 3 * PH:4 * PH])
            c = f_g * c + i_g * g_g
            h = o_g * jnp.tanh(c)
        return h, c

    z = jnp.zeros((B, PH), F32)
    hf, _ = jax.lax.fori_loop(0, N // 4, lstm_step4, (z, z))

    gb = _mm(hct_ref[N - 1], wib_ref[...]) + bb_ref[...]
    cb = jax.nn.sigmoid(gb[:, 0:PH]) * jnp.tanh(gb[:, 2 * PH:3 * PH])
    hb = jax.nn.sigmoid(gb[:, 3 * PH:4 * PH]) * jnp.tanh(cb)

    hend = jnp.concatenate([hf, hb], axis=1)
    gi = _mm(hend, wig_ref[...]) + big_ref[...]

    whg = whg_ref[...]
    bhg = bhg_ref[...]

    def gru_step4(i4, h):
        for u in range(4):
            gh = _mm(h, whg) + bhg
            r = jax.nn.sigmoid(gi[:, 0:PH] + gh[:, 0:PH])
            zg = jax.nn.sigmoid(gi[:, PH:2 * PH] + gh[:, PH:2 * PH])
            nc = jnp.tanh(gi[:, 2 * PH:3 * PH] + r * gh[:, 2 * PH:3 * PH])
            h = (1.0 - zg) * nc + zg * h
            hall_ref[i4 * 4 + u] = h
        return h

    jax.lax.fori_loop(0, N // 4, gru_step4, z)

    wfc = wfc_ref[...]
    bfc = bfc_ref[...]
    for tc in range(N // TC):
        blk = hall_ref[tc * TC:(tc + 1) * TC]
        a = blk.reshape(TC * B, PH)
        out_ref[tc * TC:(tc + 1) * TC] = (
            _mm(a, wfc) + bfc).reshape(TC, B, K)


def _run_rnn(hct, wif, whf, bf, wib, bb, wig, big, whg, bhg, wfc, bfc):
    full = lambda a: pl.BlockSpec(a.shape, lambda: tuple(0 for _ in a.shape))
    args = (hct, wif, whf, bf, wib, bb, wig, big, whg, bhg, wfc, bfc)
    return pl.pallas_call(
        _rnn_kernel,
        grid=(),
        in_specs=[full(a) for a in args],
        out_specs=pl.BlockSpec((N, B, K), lambda: (0, 0, 0)),
        out_shape=jax.ShapeDtypeStruct((N, B, K), F32),
        scratch_shapes=[pltpu.VMEM((N, B, 4 * PH), F32),
                        pltpu.VMEM((N, B, PH), F32)],
    )(*args)


def _pad_gates(w_t, n_gates, in_rows):
    in_dim = w_t.shape[1]
    out = jnp.zeros((in_rows, n_gates * PH), F32)
    for g in range(n_gates):
        out = out.at[0:in_dim, g * PH:g * PH + H].set(w_t[g * H:(g + 1) * H, :].T)
    return out


def _pad_bias(b, n_gates):
    out = jnp.zeros((1, n_gates * PH), F32)
    for g in range(n_gates):
        out = out.at[0, g * PH:g * PH + H].set(b[g * H:(g + 1) * H])
    return out


def kernel(x, fc_edge_index, tc_edge_index, conv2_W, conv2_b, conv3_W, conv3_b,
           gat_W, gat_a, gat_b, lstm_W_ih, lstm_W_hh, lstm_b_ih, lstm_b_hh,
           gru_W_ih, gru_W_hh, gru_b_ih, gru_b_hh, fc_W, fc_b):
    fc_ei = fc_edge_index[-1].astype(jnp.int32)
    tc_ei = tc_edge_index[-1].astype(jnp.int32)

    _, lcf, mf, _, lct, mt = _build_counts(fc_ei, tc_ei)

    xpad = jnp.pad(x, ((0, 0), (3, 5), (0, 0)))
    Wc = jnp.zeros((3, 7, K, K), F32)
    Wc = Wc.at[0, 3].set(jnp.eye(K, dtype=F32))
    for d in range(5):
        Wc = Wc.at[1, d + 1].set(conv2_W[:, :, d].T)
    for d in range(7):
        Wc = Wc.at[2, d].set(conv3_W[:, :, d].T)
    bc = jnp.stack([jnp.zeros_like(conv2_b), conv2_b, conv3_b]).reshape(3, 1, K)
    hs = _run_branches(xpad.astype(BF), xpad, Wc.astype(BF), bc,
                       lcf, mf, lct, mt, gat_W.astype(BF), gat_a, gat_b)

    hct = hs.transpose(2, 1, 0, 3).reshape(N, B, 3 * K)
    wif = _pad_gates(lstm_W_ih[0], 4, 3 * K)
    whf = _pad_gates(lstm_W_hh[0], 4, PH)
    bf = _pad_bias(lstm_b_ih[0] + lstm_b_hh[0], 4)
    wib = _pad_gates(lstm_W_ih[1], 4, 3 * K)
    bb = _pad_bias(lstm_b_ih[1] + lstm_b_hh[1], 4)
    wig = jnp.zeros((2 * PH, 3 * PH), F32)
    for g in range(3):
        blk = gru_W_ih[g * H:(g + 1) * H, :]
        wig = wig.at[0:H, g * PH:g * PH + H].set(blk[:, 0:H].T)
        wig = wig.at[PH:PH + H, g * PH:g * PH + H].set(blk[:, H:2 * H].T)
    big = _pad_bias(gru_b_ih, 3)
    whg = _pad_gates(gru_W_hh, 3, PH)
    bhg = _pad_bias(gru_b_hh, 3)
    wfc = jnp.zeros((PH, K), F32).at[0:H, :].set(fc_W.T)
    bfc = fc_b.reshape(1, K)
    outt = _run_rnn(hct, wif.astype(BF), whf.astype(BF), bf, wib.astype(BF),
                    bb, wig.astype(BF), big, whg.astype(BF), bhg,
                    wfc.astype(BF), bfc)
    return outt.transpose(1, 0, 2)
